# Initial kernel scaffold; baseline (speedup 1.0000x reference)
import jax
import jax.numpy as jnp
from jax.experimental import pallas as pl


def kernel(x, fc_weight, bias):
    raise NotImplementedError("write your pallas kernel here")


# speedup vs baseline: 1.3291x; 1.3291x over previous
import jax
import jax.numpy as jnp
from jax import lax
from jax.experimental import pallas as pl
from jax.experimental.pallas import tpu as pltpu
from jax.experimental.pallas import tpu_sc as plsc

B = 16384
F = 26
NC = 2
NS = 16
NW = NC * NS
BPW = B // NW
CHUNK = 128
NCH = BPW // CHUNK
NJ = F * NCH
GRP = 8


def _body(xr_hbm, tab_hbm, bias_hbm, out_hbm, idx_v, buf_v, acc_v, bias_v, sem):
    cid = lax.axis_index("c")
    sid = lax.axis_index("s")
    wid = sid * NC + cid

    pltpu.sync_copy(xr_hbm.at[wid], idx_v)
    pltpu.sync_copy(bias_hbm, bias_v)
    binit = bias_v[...]

    def fire(g):
        cps = []
        for jj in range(GRP):
            j = g * GRP + jj
            cps.append(
                pltpu.async_copy(
                    tab_hbm.at[idx_v.at[j]],
                    buf_v.at[j],
                    sem,
                )
            )
        return cps

    prev = None
    for g in range(NJ // GRP):
        cur = fire(g)
        if prev is not None:
            for cp in prev:
                cp.wait()
        prev = cur
    for cp in prev:
        cp.wait()

    for c in range(NCH):
        for g in range(CHUNK // 16):
            acc16 = binit
            for f in range(F):
                acc16 = acc16 + buf_v[f * NCH + c, pl.ds(g * 16, 16)]
            acc_v[pl.ds(c * CHUNK + g * 16, 16)] = acc16

    pltpu.sync_copy(acc_v, out_hbm.at[pl.ds(wid * BPW, BPW)])


@jax.jit
def _linear_features(xr, tab, bias):
    mesh = plsc.VectorSubcoreMesh(core_axis_name="c", subcore_axis_name="s")
    return pl.kernel(
        _body,
        out_type=jax.ShapeDtypeStruct((B,), jnp.float32),
        mesh=mesh,
        scratch_types=[
            pltpu.VMEM((NJ, CHUNK), jnp.int32),
            pltpu.VMEM((NJ, CHUNK), jnp.float32),
            pltpu.VMEM((BPW,), jnp.float32),
            pltpu.VMEM((16,), jnp.float32),
            pltpu.SemaphoreType.DMA,
        ],
    )(xr, tab, bias)


def kernel(x, fc_weight, bias):
    xr = (
        x.astype(jnp.int32)
        .T.reshape(F, NW, BPW)
        .transpose(1, 0, 2)
        .reshape(NW, NJ, CHUNK)
    )
    out = _linear_features(
        xr, fc_weight.reshape(-1), jnp.broadcast_to(bias, (16,))
    )
    return out.reshape(B, 1)
